# Initial kernel scaffold; baseline (speedup 1.0000x reference)
import jax
import jax.numpy as jnp
from jax.experimental import pallas as pl


def kernel(x, edge_index, W1, b1, W2, b2, W3, b3, fcW, fcb):
    raise NotImplementedError("write your pallas kernel here")


# speedup vs baseline: 16.4597x; 16.4597x over previous
import functools

import jax
import jax.numpy as jnp
from jax import lax
from jax.experimental import pallas as pl
from jax.experimental.pallas import tpu as pltpu
from jax.experimental.pallas import tpu_sc as plsc

_N = 10000
_NPAD = 10240
_B = 120
_NW = 32
_NC = 2
_ROWS_PER_SID = _NPAD // 16


def _make_scatter(d, e_pad):
  epw = e_pad // _NW
  nblk = epw // _B
  mesh = plsc.VectorSubcoreMesh(core_axis_name="c", subcore_axis_name="s")

  @functools.partial(
      pl.kernel,
      mesh=mesh,
      out_type=jax.ShapeDtypeStruct((_NC, _NPAD, d), jnp.float32),
      scratch_types=[
          pltpu.VMEM_SHARED((_NPAD, d), jnp.float32),
          pltpu.VMEM((_B,), jnp.int32),
          pltpu.VMEM((_B,), jnp.int32),
          pltpu.VMEM((_B, d), jnp.float32),
          pltpu.SemaphoreType.DMA,
      ],
  )
  def k(h_hbm, row_hbm, col_hbm, z_hbm, out_hbm, acc, ridx, cidx, rows, sem):
    cid = lax.axis_index("c")
    sid = lax.axis_index("s")
    wid = sid * _NC + cid
    pltpu.sync_copy(z_hbm, acc.at[pl.ds(sid * _ROWS_PER_SID, _ROWS_PER_SID)])
    plsc.subcore_barrier()
    base = wid * epw

    def body(b, carry):
      off = base + b * _B
      pltpu.sync_copy(row_hbm.at[pl.ds(off, _B)], ridx)
      pltpu.sync_copy(col_hbm.at[pl.ds(off, _B)], cidx)
      pltpu.async_copy(h_hbm.at[ridx], rows, sem).wait()
      pltpu.sync_copy(rows, acc.at[cidx], add=True)
      return carry

    lax.fori_loop(0, nblk, body, 0)
    plsc.subcore_barrier()
    sl = pl.ds(sid * _ROWS_PER_SID, _ROWS_PER_SID)
    pltpu.sync_copy(acc.at[sl], out_hbm.at[cid, sl])

  return k


def _combine(partials, sa, sb):
  d = partials.shape[-1]
  blk = 512

  def body(p_ref, a_ref, b_ref, v_ref, y_ref):
    s = p_ref[0] + p_ref[1]
    v_ref[...] = s * a_ref[...]
    y_ref[...] = s * b_ref[...]

  return pl.pallas_call(
      body,
      grid=(_NPAD // blk,),
      in_specs=[
          pl.BlockSpec((2, blk, d), lambda i: (0, i, 0)),
          pl.BlockSpec((blk, 1), lambda i: (i, 0)),
          pl.BlockSpec((blk, 1), lambda i: (i, 0)),
      ],
      out_specs=[
          pl.BlockSpec((blk, d), lambda i: (i, 0)),
          pl.BlockSpec((blk, d), lambda i: (i, 0)),
      ],
      out_shape=[jax.ShapeDtypeStruct((_NPAD, d), jnp.float32)] * 2,
  )(partials, sa, sb)


def _weights_kernel(W1, W2, W3, b1, b2, b3, fcW, fcb):
  def body(w1, w2, w3, bb1, bb2, bb3, fw, fb, out):
    f = fw[...]
    ats = []
    c1 = []
    c2 = []
    c3sum = jnp.zeros((128,), jnp.float32)
    for k in range(3):
      fw3 = jnp.dot(f, w3[k], preferred_element_type=jnp.float32)
      fw32 = jnp.dot(fw3, w2[k], preferred_element_type=jnp.float32)
      at = jnp.dot(fw32, w1[k], preferred_element_type=jnp.float32) / 3.0
      ats.append(at)
      c1.append(jnp.dot(fw32, bb1[k], preferred_element_type=jnp.float32) / 3.0)
      c2.append(jnp.dot(fw3, bb2[k], preferred_element_type=jnp.float32) / 3.0)
      c3sum = c3sum + jnp.dot(f, bb3[k], preferred_element_type=jnp.float32) / 3.0
    const_row = c3sum + fb[...]
    cols = jnp.stack(
        [const_row, c2[0], c1[0] + c2[1], c2[2], c1[1], c1[2]], axis=1)
    out[...] = jnp.concatenate(
        [ats[0], ats[1], ats[2], cols, jnp.zeros((128, 122), jnp.float32)],
        axis=1)

  return pl.pallas_call(
      body,
      out_shape=jax.ShapeDtypeStruct((128, 512), jnp.float32),
  )(W1, W2, W3, b1, b2, b3, fcW, fcb)


def _matmul(ycat, wbigT):
  blk = 512

  def body(x_ref, w_ref, o_ref):
    o_ref[...] = lax.dot_general(
        x_ref[...], w_ref[...], (((1,), (1,)), ((), ())),
        preferred_element_type=jnp.float32)

  return pl.pallas_call(
      body,
      grid=(_NPAD // blk,),
      in_specs=[
          pl.BlockSpec((blk, 512), lambda i: (i, 0)),
          pl.BlockSpec((128, 512), lambda i: (0, 0)),
      ],
      out_specs=pl.BlockSpec((blk, 128), lambda i: (i, 0)),
      out_shape=jax.ShapeDtypeStruct((_NPAD, 128), jnp.float32),
  )(ycat, wbigT)


def kernel(x, edge_index, W1, b1, W2, b2, W3, b3, fcW, fcb):
  n = x.shape[0]
  e = edge_index.shape[1]
  e_full = e + n
  e_pad = ((e_full + _NW * _B - 1) // (_NW * _B)) * (_NW * _B)
  npad_extra = e_pad - e_full
  loops = jnp.arange(n, dtype=jnp.int32)
  pad_col = _N + 100 + (jnp.arange(npad_extra, dtype=jnp.int32) % 64)
  row_full = jnp.concatenate(
      [edge_index[0], loops, jnp.zeros((npad_extra,), jnp.int32)])
  col_full = jnp.concatenate([edge_index[1], loops, pad_col])

  z128 = jnp.zeros((_ROWS_PER_SID, 128), jnp.float32)

  ones128 = jnp.ones((_NPAD, 128), jnp.float32)
  prop = _make_scatter(128, e_pad)
  degp = prop(ones128, row_full, col_full, z128)
  deg = degp[0, :_N, 0] + degp[1, :_N, 0]
  dinvh = jnp.where(deg > 0, deg ** -0.5, 0.0)
  dinv2 = jnp.where(deg > 0, 1.0 / deg, 0.0)
  dinvh_c = jnp.pad(dinvh, (0, _NPAD - _N))[:, None]
  dinv2_c = jnp.pad(dinv2, (0, _NPAD - _N))[:, None]

  xa = jnp.zeros((_NPAD, 128), jnp.float32).at[:_N, :].set(x)
  v = xa * dinvh_c
  ys = {}
  for j in range(1, 10):
    partials = prop(v, row_full, col_full, z128)
    v, yj = _combine(partials, dinv2_c, dinvh_c)
    if j in (3, 6, 9):
      ys[j] = yj

  ones_col = jnp.zeros((_NPAD, 1), jnp.float32).at[:_N, 0].set(1.0)
  ycat = jnp.concatenate(
      [ys[3], ys[6], ys[9], ones_col,
       jnp.zeros((_NPAD, 127), jnp.float32)], axis=1)

  wbigT = _weights_kernel(W1, W2, W3, b1, b2, b3, fcW, fcb)
  out = _matmul(ycat, wbigT)
  return out[:_N]


# speedup vs baseline: 22.2150x; 1.3497x over previous
import functools

import jax
import jax.numpy as jnp
from jax import lax
from jax.experimental import pallas as pl
from jax.experimental.pallas import tpu as pltpu
from jax.experimental.pallas import tpu_sc as plsc

_N = 10000
_NPAD = 10240
_B = 120
_NW = 32
_NC = 2
_ROWS_PER_SID = _NPAD // 16


def _make_scatter(d, e_pad):
  epw = e_pad // _NW
  nblk = epw // _B
  mesh = plsc.VectorSubcoreMesh(core_axis_name="c", subcore_axis_name="s")

  @functools.partial(
      pl.kernel,
      mesh=mesh,
      out_type=jax.ShapeDtypeStruct((_NC, _NPAD, d), jnp.float32),
      scratch_types=[
          pltpu.VMEM_SHARED((_NPAD, d), jnp.float32),
          pltpu.VMEM((_B,), jnp.int32),
          pltpu.VMEM((_B,), jnp.int32),
          pltpu.VMEM((_B,), jnp.int32),
          pltpu.VMEM((_B,), jnp.int32),
          pltpu.VMEM((_B, d), jnp.float32),
          pltpu.VMEM((_B, d), jnp.float32),
          pltpu.SemaphoreType.DMA,
          pltpu.SemaphoreType.DMA,
      ],
  )
  def k(h_hbm, row_hbm, col_hbm, z_hbm, out_hbm, acc,
        ridx0, ridx1, cidx0, cidx1, rows0, rows1, sem0, sem1):
    cid = lax.axis_index("c")
    sid = lax.axis_index("s")
    wid = sid * _NC + cid
    pltpu.sync_copy(z_hbm, acc.at[pl.ds(sid * _ROWS_PER_SID, _ROWS_PER_SID)])
    plsc.subcore_barrier()
    base = wid * epw
    ridx = (ridx0, ridx1)
    cidx = (cidx0, cidx1)
    rows = (rows0, rows1)
    sems = (sem0, sem1)

    def body(g, carry):
      hs = []
      for s in range(2):
        off = base + (g * 2 + s) * _B
        pltpu.sync_copy(row_hbm.at[pl.ds(off, _B)], ridx[s])
        pltpu.sync_copy(col_hbm.at[pl.ds(off, _B)], cidx[s])
        hs.append(pltpu.async_copy(h_hbm.at[ridx[s]], rows[s], sems[s]))
      for s in range(2):
        hs[s].wait()
        pltpu.sync_copy(rows[s], acc.at[cidx[s]], add=True)
      return carry

    lax.fori_loop(0, nblk // 2, body, 0)
    plsc.subcore_barrier()
    sl = pl.ds(sid * _ROWS_PER_SID, _ROWS_PER_SID)
    pltpu.sync_copy(acc.at[sl], out_hbm.at[cid, sl])

  return k


def _combine(partials, sa, sb):
  d = partials.shape[-1]
  blk = 512

  def body(p_ref, a_ref, b_ref, v_ref, y_ref):
    s = p_ref[0] + p_ref[1]
    v_ref[...] = s * a_ref[...]
    y_ref[...] = s * b_ref[...]

  return pl.pallas_call(
      body,
      grid=(_NPAD // blk,),
      in_specs=[
          pl.BlockSpec((2, blk, d), lambda i: (0, i, 0)),
          pl.BlockSpec((blk, 1), lambda i: (i, 0)),
          pl.BlockSpec((blk, 1), lambda i: (i, 0)),
      ],
      out_specs=[
          pl.BlockSpec((blk, d), lambda i: (i, 0)),
          pl.BlockSpec((blk, d), lambda i: (i, 0)),
      ],
      out_shape=[jax.ShapeDtypeStruct((_NPAD, d), jnp.float32)] * 2,
  )(partials, sa, sb)


def _weights_kernel(W1, W2, W3, b1, b2, b3, fcW, fcb):
  def body(w1, w2, w3, bb1, bb2, bb3, fw, fb, out):
    f = fw[...]
    ats = []
    c1 = []
    c2 = []
    c3sum = jnp.zeros((128,), jnp.float32)
    for k in range(3):
      fw3 = jnp.dot(f, w3[k], preferred_element_type=jnp.float32)
      fw32 = jnp.dot(fw3, w2[k], preferred_element_type=jnp.float32)
      at = jnp.dot(fw32, w1[k], preferred_element_type=jnp.float32) / 3.0
      ats.append(at)
      c1.append(jnp.dot(fw32, bb1[k], preferred_element_type=jnp.float32) / 3.0)
      c2.append(jnp.dot(fw3, bb2[k], preferred_element_type=jnp.float32) / 3.0)
      c3sum = c3sum + jnp.dot(f, bb3[k], preferred_element_type=jnp.float32) / 3.0
    const_row = c3sum + fb[...]
    cols = jnp.stack(
        [const_row, c2[0], c1[0] + c2[1], c2[2], c1[1], c1[2]], axis=1)
    out[...] = jnp.concatenate(
        [ats[0], ats[1], ats[2], cols, jnp.zeros((128, 122), jnp.float32)],
        axis=1)

  return pl.pallas_call(
      body,
      out_shape=jax.ShapeDtypeStruct((128, 512), jnp.float32),
  )(W1, W2, W3, b1, b2, b3, fcW, fcb)


def _matmul(ycat, wbigT):
  blk = 512

  def body(x_ref, w_ref, o_ref):
    o_ref[...] = lax.dot_general(
        x_ref[...], w_ref[...], (((1,), (1,)), ((), ())),
        preferred_element_type=jnp.float32)

  return pl.pallas_call(
      body,
      grid=(_NPAD // blk,),
      in_specs=[
          pl.BlockSpec((blk, 512), lambda i: (i, 0)),
          pl.BlockSpec((128, 512), lambda i: (0, 0)),
      ],
      out_specs=pl.BlockSpec((blk, 128), lambda i: (i, 0)),
      out_shape=jax.ShapeDtypeStruct((_NPAD, 128), jnp.float32),
  )(ycat, wbigT)


def kernel(x, edge_index, W1, b1, W2, b2, W3, b3, fcW, fcb):
  n = x.shape[0]
  e = edge_index.shape[1]
  e_full = e + n
  e_pad = ((e_full + _NW * _B - 1) // (_NW * _B)) * (_NW * _B)
  npad_extra = e_pad - e_full
  loops = jnp.arange(n, dtype=jnp.int32)
  pad_col = _N + 100 + (jnp.arange(npad_extra, dtype=jnp.int32) % 64)
  row_full = jnp.concatenate(
      [edge_index[0], loops, jnp.zeros((npad_extra,), jnp.int32)])
  col_full = jnp.concatenate([edge_index[1], loops, pad_col])

  z128 = jnp.zeros((_ROWS_PER_SID, 128), jnp.float32)

  ones128 = jnp.ones((_NPAD, 128), jnp.float32)
  prop = _make_scatter(128, e_pad)
  degp = prop(ones128, row_full, col_full, z128)
  deg = degp[0, :_N, 0] + degp[1, :_N, 0]
  dinvh = jnp.where(deg > 0, deg ** -0.5, 0.0)
  dinv2 = jnp.where(deg > 0, 1.0 / deg, 0.0)
  dinvh_c = jnp.pad(dinvh, (0, _NPAD - _N))[:, None]
  dinv2_c = jnp.pad(dinv2, (0, _NPAD - _N))[:, None]

  xa = jnp.zeros((_NPAD, 128), jnp.float32).at[:_N, :].set(x)
  v = xa * dinvh_c
  ys = {}
  for j in range(1, 10):
    partials = prop(v, row_full, col_full, z128)
    v, yj = _combine(partials, dinv2_c, dinvh_c)
    if j in (3, 6, 9):
      ys[j] = yj

  ones_col = jnp.zeros((_NPAD, 1), jnp.float32).at[:_N, 0].set(1.0)
  ycat = jnp.concatenate(
      [ys[3], ys[6], ys[9], ones_col,
       jnp.zeros((_NPAD, 127), jnp.float32)], axis=1)

  wbigT = _weights_kernel(W1, W2, W3, b1, b2, b3, fcW, fcb)
  out = _matmul(ycat, wbigT)
  return out[:_N]


# speedup vs baseline: 22.2213x; 1.0003x over previous
import functools

import jax
import jax.numpy as jnp
from jax import lax
from jax.experimental import pallas as pl
from jax.experimental.pallas import tpu as pltpu
from jax.experimental.pallas import tpu_sc as plsc

_N = 10000
_NPAD = 10240
_B = 120
_NW = 32
_NC = 2
_ROWS_PER_SID = _NPAD // 16
_SLOTS = 2


def _make_scatter(d, e_pad):
  epw = e_pad // _NW
  nblk = epw // _B
  mesh = plsc.VectorSubcoreMesh(core_axis_name="c", subcore_axis_name="s")

  @functools.partial(
      pl.kernel,
      mesh=mesh,
      out_type=jax.ShapeDtypeStruct((_NC, _NPAD, d), jnp.float32),
      scratch_types=(
          [pltpu.VMEM_SHARED((_NPAD, d), jnp.float32)]
          + [pltpu.VMEM((_B,), jnp.int32)] * (2 * _SLOTS)
          + [pltpu.VMEM((_B, d), jnp.float32)] * _SLOTS
          + [pltpu.SemaphoreType.DMA] * _SLOTS
      ),
  )
  def k(h_hbm, row_hbm, col_hbm, z_hbm, out_hbm, acc, *scr):
    ridx = scr[:_SLOTS]
    cidx = scr[_SLOTS:2 * _SLOTS]
    rows = scr[2 * _SLOTS:3 * _SLOTS]
    sems = scr[3 * _SLOTS:]
    cid = lax.axis_index("c")
    sid = lax.axis_index("s")
    wid = sid * _NC + cid
    pltpu.sync_copy(z_hbm, acc.at[pl.ds(sid * _ROWS_PER_SID, _ROWS_PER_SID)])
    plsc.subcore_barrier()
    base = wid * epw

    def body(g, carry):
      hs = []
      for s in range(_SLOTS):
        off = base + (g * _SLOTS + s) * _B
        pltpu.sync_copy(row_hbm.at[pl.ds(off, _B)], ridx[s])
        pltpu.sync_copy(col_hbm.at[pl.ds(off, _B)], cidx[s])
        hs.append(pltpu.async_copy(h_hbm.at[ridx[s]], rows[s], sems[s]))
      for s in range(_SLOTS):
        hs[s].wait()
        pltpu.sync_copy(rows[s], acc.at[cidx[s]], add=True)
      return carry

    lax.fori_loop(0, nblk // _SLOTS, body, 0)
    plsc.subcore_barrier()
    sl = pl.ds(sid * _ROWS_PER_SID, _ROWS_PER_SID)
    pltpu.sync_copy(acc.at[sl], out_hbm.at[cid, sl])

  return k


def _combine(partials, sa, sb):
  d = partials.shape[-1]
  blk = 512

  def body(p_ref, a_ref, b_ref, v_ref, y_ref):
    s = p_ref[0] + p_ref[1]
    v_ref[...] = s * a_ref[...]
    y_ref[...] = s * b_ref[...]

  return pl.pallas_call(
      body,
      grid=(_NPAD // blk,),
      in_specs=[
          pl.BlockSpec((2, blk, d), lambda i: (0, i, 0)),
          pl.BlockSpec((blk, 1), lambda i: (i, 0)),
          pl.BlockSpec((blk, 1), lambda i: (i, 0)),
      ],
      out_specs=[
          pl.BlockSpec((blk, d), lambda i: (i, 0)),
          pl.BlockSpec((blk, d), lambda i: (i, 0)),
      ],
      out_shape=[jax.ShapeDtypeStruct((_NPAD, d), jnp.float32)] * 2,
  )(partials, sa, sb)


def _weights_kernel(W1, W2, W3, b1, b2, b3, fcW, fcb):
  def body(w1, w2, w3, bb1, bb2, bb3, fw, fb, out):
    f = fw[...]
    ats = []
    c1 = []
    c2 = []
    c3sum = jnp.zeros((128,), jnp.float32)
    for k in range(3):
      fw3 = jnp.dot(f, w3[k], preferred_element_type=jnp.float32)
      fw32 = jnp.dot(fw3, w2[k], preferred_element_type=jnp.float32)
      at = jnp.dot(fw32, w1[k], preferred_element_type=jnp.float32) / 3.0
      ats.append(at)
      c1.append(jnp.dot(fw32, bb1[k], preferred_element_type=jnp.float32) / 3.0)
      c2.append(jnp.dot(fw3, bb2[k], preferred_element_type=jnp.float32) / 3.0)
      c3sum = c3sum + jnp.dot(f, bb3[k], preferred_element_type=jnp.float32) / 3.0
    const_row = c3sum + fb[...]
    cols = jnp.stack(
        [const_row, c2[0], c1[0] + c2[1], c2[2], c1[1], c1[2]], axis=1)
    out[...] = jnp.concatenate(
        [ats[0], ats[1], ats[2], cols, jnp.zeros((128, 122), jnp.float32)],
        axis=1)

  return pl.pallas_call(
      body,
      out_shape=jax.ShapeDtypeStruct((128, 512), jnp.float32),
  )(W1, W2, W3, b1, b2, b3, fcW, fcb)


def _matmul(ycat, wbigT):
  blk = 512

  def body(x_ref, w_ref, o_ref):
    o_ref[...] = lax.dot_general(
        x_ref[...], w_ref[...], (((1,), (1,)), ((), ())),
        preferred_element_type=jnp.float32)

  return pl.pallas_call(
      body,
      grid=(_NPAD // blk,),
      in_specs=[
          pl.BlockSpec((blk, 512), lambda i: (i, 0)),
          pl.BlockSpec((128, 512), lambda i: (0, 0)),
      ],
      out_specs=pl.BlockSpec((blk, 128), lambda i: (i, 0)),
      out_shape=jax.ShapeDtypeStruct((_NPAD, 128), jnp.float32),
  )(ycat, wbigT)


def kernel(x, edge_index, W1, b1, W2, b2, W3, b3, fcW, fcb):
  n = x.shape[0]
  e = edge_index.shape[1]
  e_full = e + n
  quant = _NW * _B * _SLOTS
  e_pad = ((e_full + quant - 1) // quant) * quant
  npad_extra = e_pad - e_full
  loops = jnp.arange(n, dtype=jnp.int32)
  pad_col = _N + 100 + (jnp.arange(npad_extra, dtype=jnp.int32) % 64)
  row_full = jnp.concatenate(
      [edge_index[0], loops, jnp.zeros((npad_extra,), jnp.int32)])
  col_full = jnp.concatenate([edge_index[1], loops, pad_col])

  z128 = jnp.zeros((_ROWS_PER_SID, 128), jnp.float32)

  ones128 = jnp.ones((_NPAD, 128), jnp.float32)
  prop = _make_scatter(128, e_pad)
  degp = prop(ones128, row_full, col_full, z128)
  deg = degp[0, :_N, 0] + degp[1, :_N, 0]
  dinvh = jnp.where(deg > 0, deg ** -0.5, 0.0)
  dinv2 = jnp.where(deg > 0, 1.0 / deg, 0.0)
  dinvh_c = jnp.pad(dinvh, (0, _NPAD - _N))[:, None]
  dinv2_c = jnp.pad(dinv2, (0, _NPAD - _N))[:, None]

  xa = jnp.zeros((_NPAD, 128), jnp.float32).at[:_N, :].set(x)
  v = xa * dinvh_c
  ys = {}
  for j in range(1, 10):
    partials = prop(v, row_full, col_full, z128)
    v, yj = _combine(partials, dinv2_c, dinvh_c)
    if j in (3, 6, 9):
      ys[j] = yj

  ones_col = jnp.zeros((_NPAD, 1), jnp.float32).at[:_N, 0].set(1.0)
  ycat = jnp.concatenate(
      [ys[3], ys[6], ys[9], ones_col,
       jnp.zeros((_NPAD, 127), jnp.float32)], axis=1)

  wbigT = _weights_kernel(W1, W2, W3, b1, b2, b3, fcW, fcb)
  out = _matmul(ycat, wbigT)
  return out[:_N]
